# Initial kernel scaffold; baseline (speedup 1.0000x reference)
import jax
import jax.numpy as jnp
from jax.experimental import pallas as pl


def kernel(logits, prev_tokens):
    raise NotImplementedError("write your pallas kernel here")


# speedup vs baseline: 2.2254x; 2.2254x over previous
import jax
import jax.numpy as jnp
from jax.experimental import pallas as pl
from jax.experimental.pallas import tpu as pltpu

_NBEAMS = 4
_BATCH = 32
_V = 100000
_EOS = 2
_NEG = -1e9
_RG = 8
_NROWS = _NBEAMS * _BATCH


def _scan_body(x_ref, pt_ref, vals_ref, idx_ref):
    i = pl.program_id(0)
    cols = jax.lax.broadcasted_iota(jnp.int32, (_RG, _V), 1)

    @pl.when(i == 0)
    def _():
        x = x_ref[...]
        banned = jnp.zeros((_RG, _V), jnp.bool_)
        for j in range(_BATCH):
            banned = jnp.logical_or(banned, cols == pt_ref[:, j][:, None])
        x_ref[...] = jnp.where(banned, _NEG, x)

    x = jnp.where(cols == _EOS, _NEG, x_ref[...])
    m = jnp.max(x, axis=1, keepdims=True)
    s = jnp.sum(jnp.exp(x - m), axis=1, keepdims=True)
    shift = m + jnp.log(s)

    vals = []
    idxs = []
    for _ in range(4):
        bm = jnp.max(x, axis=1, keepdims=True)
        bi = jnp.min(jnp.where(x == bm, cols, _V), axis=1, keepdims=True)
        vals.append(bm - shift)
        idxs.append(bi)
        x = jnp.where(cols == bi, _NEG, x)
    vals_ref[...] = jnp.concatenate(vals, axis=1)
    idx_ref[...] = jnp.concatenate(idxs, axis=1)


def _merge_body(v_ref, ix_ref, p_ref, tok_ref, beam_ref):
    v = v_ref[...]
    vocab = ix_ref[...]
    lane = jax.lax.broadcasted_iota(jnp.int32, (_BATCH, 16), 1)
    combo = (lane // 4) * _V + vocab
    big = jnp.int32(2**31 - 1)
    ps, toks, beams = [], [], []
    for _ in range(4):
        bm = jnp.max(v, axis=1, keepdims=True)
        bc = jnp.min(jnp.where(v == bm, combo, big), axis=1, keepdims=True)
        ps.append(bm)
        toks.append(bc % _V)
        beams.append(bc // _V)
        v = jnp.where(combo == bc, _NEG, v)
    p_ref[...] = jnp.concatenate(ps, axis=1)
    tok_ref[...] = jnp.concatenate(toks, axis=1)
    beam_ref[...] = jnp.concatenate(beams, axis=1)


def kernel(logits, prev_tokens):
    x = logits.reshape(_NROWS, _V)
    pt = jnp.pad(prev_tokens.astype(jnp.int32), ((0, _RG - _NBEAMS), (0, 0)),
                 constant_values=-1)

    vals, idxs = pl.pallas_call(
        _scan_body,
        grid=(_NROWS // _RG,),
        in_specs=[
            pl.BlockSpec((_RG, _V), lambda i: (i, 0)),
            pl.BlockSpec((_RG, _BATCH), lambda i: (0, 0)),
        ],
        out_specs=[
            pl.BlockSpec((_RG, 4), lambda i: (i, 0)),
            pl.BlockSpec((_RG, 4), lambda i: (i, 0)),
        ],
        out_shape=[
            jax.ShapeDtypeStruct((_NROWS, 4), jnp.float32),
            jax.ShapeDtypeStruct((_NROWS, 4), jnp.int32),
        ],
        compiler_params=pltpu.CompilerParams(
            dimension_semantics=("arbitrary",)),
    )(x, pt)

    cv = vals.reshape(_NBEAMS, _BATCH, 4).transpose(1, 0, 2).reshape(_BATCH, 16)
    ci = idxs.reshape(_NBEAMS, _BATCH, 4).transpose(1, 0, 2).reshape(_BATCH, 16)

    probs, toks, beams = pl.pallas_call(
        _merge_body,
        out_shape=[
            jax.ShapeDtypeStruct((_BATCH, 4), jnp.float32),
            jax.ShapeDtypeStruct((_BATCH, 4), jnp.int32),
            jax.ShapeDtypeStruct((_BATCH, 4), jnp.int32),
        ],
    )(cv, ci)

    new_ids = toks.T.reshape(-1, 1)
    return new_ids, probs.T, beams.T


# speedup vs baseline: 3.7054x; 1.6650x over previous
import jax
import jax.numpy as jnp
from jax.experimental import pallas as pl
from jax.experimental.pallas import tpu as pltpu

_NBEAMS = 4
_BATCH = 32
_V = 100000
_EOS = 2
_NEG = -1e9
_NROWS = _NBEAMS * _BATCH
_VB = 10000
_NSTEPS = _V // _VB


def _ce(va, ia, vb, ib):
    sw = (vb > va) | ((vb == va) & (ib < ia))
    return (jnp.where(sw, vb, va), jnp.where(sw, ib, ia),
            jnp.where(sw, va, vb), jnp.where(sw, ia, ib))


def _scan_body(xt_ref, pt_ref, vals_ref, idx_ref, s_ref, tv_ref, ti_ref):
    i = pl.program_id(0)
    f32 = jnp.float32

    @pl.when(i == 0)
    def _():
        xt_ref[_EOS:_EOS + 1, :] = jnp.full((1, _NROWS), _NEG, f32)
        s_ref[...] = jnp.zeros((8, _NROWS), f32)
        tv_ref[...] = jnp.full((8, _NROWS), _NEG, f32)
        ti_ref[...] = jnp.full((8, _NROWS), 2**30, jnp.int32)

    sub8 = jax.lax.broadcasted_iota(jnp.int32, (8, _NROWS), 0)
    lane = jax.lax.broadcasted_iota(jnp.int32, (8, _NROWS), 1)
    for r in range(_NBEAMS):
        lane_is_r = lane == r
        for j in range(_BATCH):
            pos = pt_ref[r, j] - i * _VB
            q = jnp.clip((pos // 8) * 8, 0, _VB - 8)
            q = pl.multiple_of(q, 8)
            rem = pos - q
            seg = xt_ref[pl.ds(q, 8), :]
            xt_ref[pl.ds(q, 8), :] = jnp.where(
                (sub8 == rem) & lane_is_r, _NEG, seg)

    x = xt_ref[...]
    vidx = (jax.lax.broadcasted_iota(jnp.int32, (_VB, _NROWS), 0)
            + i * _VB)
    s_ref[0:1, :] = s_ref[0:1, :] + jnp.sum(jnp.exp(x), axis=0,
                                            keepdims=True)
    big = jnp.int32(2**30)
    bv, bi_ = [], []
    cbm = jnp.max(x, axis=0, keepdims=True)
    for k in range(4):
        bi = jnp.min(jnp.where(x == cbm, vidx, big), axis=0, keepdims=True)
        bv.append(cbm)
        bi_.append(bi)
        if k < 3:
            x = jnp.where(vidx == bi, _NEG, x)
            cbm = jnp.max(x, axis=0, keepdims=True)

    h = []
    for k in range(4):
        hv, hi, _, _ = _ce(tv_ref[k:k + 1, :], ti_ref[k:k + 1, :],
                           bv[3 - k], bi_[3 - k])
        h.append((hv, hi))
    for a, b in ((0, 2), (1, 3), (0, 1), (2, 3)):
        hv, hi, lv, li = _ce(h[a][0], h[a][1], h[b][0], h[b][1])
        h[a] = (hv, hi)
        h[b] = (lv, li)
    for k in range(4):
        tv_ref[k:k + 1, :] = h[k][0]
        ti_ref[k:k + 1, :] = h[k][1]

    @pl.when(i == _NSTEPS - 1)
    def _():
        shift = jnp.log(s_ref[0:1, :])
        pad_v = jnp.full((4, _NROWS), _NEG, f32)
        pad_i = jnp.zeros((4, _NROWS), jnp.int32)
        vals_ref[...] = jnp.concatenate(
            [tv_ref[k:k + 1, :] - shift for k in range(4)] + [pad_v], axis=0)
        idx_ref[...] = jnp.concatenate(
            [ti_ref[k:k + 1, :] for k in range(4)] + [pad_i], axis=0)


def _merge_body(v_ref, ix_ref, p_ref, tok_ref, beam_ref):
    v = v_ref[...]
    vocab = ix_ref[...]
    lane = jax.lax.broadcasted_iota(jnp.int32, (_BATCH, 16), 1)
    combo = (lane // 4) * _V + vocab
    big = jnp.int32(2**31 - 1)
    ps, toks, beams = [], [], []
    for _ in range(4):
        bm = jnp.max(v, axis=1, keepdims=True)
        bc = jnp.min(jnp.where(v == bm, combo, big), axis=1, keepdims=True)
        ps.append(bm)
        toks.append(bc % _V)
        beams.append(bc // _V)
        v = jnp.where(combo == bc, _NEG, v)
    p_ref[...] = jnp.concatenate(ps, axis=1)
    tok_ref[...] = jnp.concatenate(toks, axis=1)
    beam_ref[...] = jnp.concatenate(beams, axis=1)


def kernel(logits, prev_tokens):
    xt = jnp.transpose(logits.reshape(_NROWS, _V))
    pt = prev_tokens.astype(jnp.int32)

    vals_t, idx_t = pl.pallas_call(
        _scan_body,
        grid=(_NSTEPS,),
        in_specs=[
            pl.BlockSpec((_VB, _NROWS), lambda i: (i, 0)),
            pl.BlockSpec(memory_space=pltpu.MemorySpace.SMEM),
        ],
        out_specs=[
            pl.BlockSpec((8, _NROWS), lambda i: (0, 0)),
            pl.BlockSpec((8, _NROWS), lambda i: (0, 0)),
        ],
        out_shape=[
            jax.ShapeDtypeStruct((8, _NROWS), jnp.float32),
            jax.ShapeDtypeStruct((8, _NROWS), jnp.int32),
        ],
        scratch_shapes=[
            pltpu.VMEM((8, _NROWS), jnp.float32),
            pltpu.VMEM((8, _NROWS), jnp.float32),
            pltpu.VMEM((8, _NROWS), jnp.int32),
        ],
        compiler_params=pltpu.CompilerParams(
            dimension_semantics=("arbitrary",)),
    )(xt, pt)

    cv = (vals_t[:4].reshape(4, _NBEAMS, _BATCH).transpose(2, 1, 0)
          .reshape(_BATCH, 16))
    ci = (idx_t[:4].reshape(4, _NBEAMS, _BATCH).transpose(2, 1, 0)
          .reshape(_BATCH, 16))

    probs, toks, beams = pl.pallas_call(
        _merge_body,
        out_shape=[
            jax.ShapeDtypeStruct((_BATCH, 4), jnp.float32),
            jax.ShapeDtypeStruct((_BATCH, 4), jnp.int32),
            jax.ShapeDtypeStruct((_BATCH, 4), jnp.int32),
        ],
    )(cv, ci)

    new_ids = toks.T.reshape(-1, 1)
    return new_ids, probs.T, beams.T


# speedup vs baseline: 5.2991x; 1.4301x over previous
import jax
import jax.numpy as jnp
from jax.experimental import pallas as pl
from jax.experimental.pallas import tpu as pltpu

_NBEAMS = 4
_BATCH = 32
_V = 100000
_EOS = 2
_NEG = -1e9
_NROWS = _NBEAMS * _BATCH
_VB = 10000
_NSTEPS = _V // _VB


def _ce(va, ia, vb, ib):
    sw = (vb > va) | ((vb == va) & (ib < ia))
    return (jnp.where(sw, vb, va), jnp.where(sw, ib, ia),
            jnp.where(sw, va, vb), jnp.where(sw, ia, ib))


def _scan_body(xt_ref, pt_ref, vals_ref, idx_ref, s_ref, tv_ref, ti_ref):
    i = pl.program_id(0)
    f32 = jnp.float32

    @pl.when(i == 0)
    def _():
        xt_ref[_EOS:_EOS + 1, :] = jnp.full((1, _NROWS), _NEG, f32)
        s_ref[...] = jnp.zeros((8, _NROWS), f32)
        tv_ref[...] = jnp.full((8, _NROWS), _NEG, f32)
        ti_ref[...] = jnp.full((8, _NROWS), 2**30, jnp.int32)

    sub8 = jax.lax.broadcasted_iota(jnp.int32, (8, _NROWS), 0)
    lane = jax.lax.broadcasted_iota(jnp.int32, (8, _NROWS), 1)
    for r in range(_NBEAMS):
        lane_is_r = lane == r
        for j in range(_BATCH):
            pos = pt_ref[r, j] - i * _VB
            q = jnp.clip((pos // 8) * 8, 0, _VB - 8)
            q = pl.multiple_of(q, 8)
            rem = pos - q
            seg = xt_ref[pl.ds(q, 8), :]
            xt_ref[pl.ds(q, 8), :] = jnp.where(
                (sub8 == rem) & lane_is_r, _NEG, seg)

    x = xt_ref[...]
    vidx = jax.lax.broadcasted_iota(jnp.int32, (_VB, _NROWS), 0)
    s_ref[0:1, :] = s_ref[0:1, :] + jnp.sum(jnp.exp(x), axis=0,
                                            keepdims=True)

    ns, sl = 10, _VB // 10
    xs = [x[j * sl:(j + 1) * sl] for j in range(ns)]
    vs = [vidx[j * sl:(j + 1) * sl] for j in range(ns)]

    def _tree(parts, op):
        while len(parts) > 1:
            parts = [op(parts[k], parts[k + 1]) if k + 1 < len(parts)
                     else parts[k] for k in range(0, len(parts), 2)]
        return parts[0]

    def _smax(chunks):
        return _tree([jnp.max(c, axis=0, keepdims=True) for c in chunks],
                     jnp.maximum)

    big = jnp.int32(2**30)
    bv, bi_ = [], []
    cbm = _smax(xs)
    for k in range(4):
        bi = _tree([jnp.min(jnp.where(xs[j] == cbm, vs[j], big), axis=0,
                            keepdims=True) for j in range(ns)],
                   jnp.minimum)
        bv.append(cbm)
        bi_.append(bi + i * _VB)
        if k < 3:
            xs = [jnp.where(vs[j] == bi, _NEG, xs[j]) for j in range(ns)]
            cbm = _smax(xs)

    h = []
    for k in range(4):
        hv, hi, _, _ = _ce(tv_ref[k:k + 1, :], ti_ref[k:k + 1, :],
                           bv[3 - k], bi_[3 - k])
        h.append((hv, hi))
    for a, b in ((0, 2), (1, 3), (0, 1), (2, 3)):
        hv, hi, lv, li = _ce(h[a][0], h[a][1], h[b][0], h[b][1])
        h[a] = (hv, hi)
        h[b] = (lv, li)
    for k in range(4):
        tv_ref[k:k + 1, :] = h[k][0]
        ti_ref[k:k + 1, :] = h[k][1]

    @pl.when(i == _NSTEPS - 1)
    def _():
        shift = jnp.log(s_ref[0:1, :])
        pad_v = jnp.full((4, _NROWS), _NEG, f32)
        pad_i = jnp.zeros((4, _NROWS), jnp.int32)
        vals_ref[...] = jnp.concatenate(
            [tv_ref[k:k + 1, :] - shift for k in range(4)] + [pad_v], axis=0)
        idx_ref[...] = jnp.concatenate(
            [ti_ref[k:k + 1, :] for k in range(4)] + [pad_i], axis=0)


def _merge_body(v_ref, ix_ref, p_ref, tok_ref, beam_ref):
    v = v_ref[...]
    vocab = ix_ref[...]
    lane = jax.lax.broadcasted_iota(jnp.int32, (_BATCH, 16), 1)
    combo = (lane // 4) * _V + vocab
    big = jnp.int32(2**31 - 1)
    ps, toks, beams = [], [], []
    for _ in range(4):
        bm = jnp.max(v, axis=1, keepdims=True)
        bc = jnp.min(jnp.where(v == bm, combo, big), axis=1, keepdims=True)
        ps.append(bm)
        toks.append(bc % _V)
        beams.append(bc // _V)
        v = jnp.where(combo == bc, _NEG, v)
    p_ref[...] = jnp.concatenate(ps, axis=1)
    tok_ref[...] = jnp.concatenate(toks, axis=1)
    beam_ref[...] = jnp.concatenate(beams, axis=1)


def kernel(logits, prev_tokens):
    xt = jnp.transpose(logits.reshape(_NROWS, _V))
    pt = prev_tokens.astype(jnp.int32)

    vals_t, idx_t = pl.pallas_call(
        _scan_body,
        grid=(_NSTEPS,),
        in_specs=[
            pl.BlockSpec((_VB, _NROWS), lambda i: (i, 0)),
            pl.BlockSpec(memory_space=pltpu.MemorySpace.SMEM),
        ],
        out_specs=[
            pl.BlockSpec((8, _NROWS), lambda i: (0, 0)),
            pl.BlockSpec((8, _NROWS), lambda i: (0, 0)),
        ],
        out_shape=[
            jax.ShapeDtypeStruct((8, _NROWS), jnp.float32),
            jax.ShapeDtypeStruct((8, _NROWS), jnp.int32),
        ],
        scratch_shapes=[
            pltpu.VMEM((8, _NROWS), jnp.float32),
            pltpu.VMEM((8, _NROWS), jnp.float32),
            pltpu.VMEM((8, _NROWS), jnp.int32),
        ],
        compiler_params=pltpu.CompilerParams(
            dimension_semantics=("arbitrary",)),
    )(xt, pt)

    cv = (vals_t[:4].reshape(4, _NBEAMS, _BATCH).transpose(2, 1, 0)
          .reshape(_BATCH, 16))
    ci = (idx_t[:4].reshape(4, _NBEAMS, _BATCH).transpose(2, 1, 0)
          .reshape(_BATCH, 16))

    probs, toks, beams = pl.pallas_call(
        _merge_body,
        out_shape=[
            jax.ShapeDtypeStruct((_BATCH, 4), jnp.float32),
            jax.ShapeDtypeStruct((_BATCH, 4), jnp.int32),
            jax.ShapeDtypeStruct((_BATCH, 4), jnp.int32),
        ],
    )(cv, ci)

    new_ids = toks.T.reshape(-1, 1)
    return new_ids, probs.T, beams.T


# speedup vs baseline: 5.4285x; 1.0244x over previous
import jax
import jax.numpy as jnp
from jax.experimental import pallas as pl
from jax.experimental.pallas import tpu as pltpu

_NBEAMS = 4
_BATCH = 32
_V = 100000
_EOS = 2
_NEG = -1e9
_NROWS = _NBEAMS * _BATCH
_VB = 20000
_NSTEPS = _V // _VB
_NS = 20
_SL = _VB // _NS


def _ce(va, ia, vb, ib):
    sw = (vb > va) | ((vb == va) & (ib < ia))
    return (jnp.where(sw, vb, va), jnp.where(sw, ib, ia),
            jnp.where(sw, va, vb), jnp.where(sw, ia, ib))


def _scan_body(xt_ref, pt_ref, vals_ref, idx_ref, s_ref, tv_ref, ti_ref):
    i = pl.program_id(0)
    f32 = jnp.float32

    @pl.when(i == 0)
    def _():
        xt_ref[_EOS:_EOS + 1, :] = jnp.full((1, _NROWS), _NEG, f32)
        s_ref[...] = jnp.zeros((8, _NROWS), f32)
        tv_ref[...] = jnp.full((8, _NROWS), _NEG, f32)
        ti_ref[...] = jnp.full((8, _NROWS), 2**30, jnp.int32)

    sub8 = jax.lax.broadcasted_iota(jnp.int32, (8, _NROWS), 0)
    lane = jax.lax.broadcasted_iota(jnp.int32, (8, _NROWS), 1)
    for r in range(_NBEAMS):
        lane_is_r = lane == r
        for j in range(_BATCH):
            pos = pt_ref[r, j] - i * _VB
            q = jnp.clip((pos // 8) * 8, 0, _VB - 8)
            q = pl.multiple_of(q, 8)
            rem = pos - q
            seg = xt_ref[pl.ds(q, 8), :]
            xt_ref[pl.ds(q, 8), :] = jnp.where(
                (sub8 == rem) & lane_is_r, _NEG, seg)

    vidx = jax.lax.broadcasted_iota(jnp.int32, (_VB, _NROWS), 0)

    ns, sl = _NS, _SL
    xs = [xt_ref[j * sl:(j + 1) * sl, :] for j in range(ns)]
    vs = [vidx[j * sl:(j + 1) * sl] for j in range(ns)]

    def _tree(parts, op):
        while len(parts) > 1:
            parts = [op(parts[k], parts[k + 1]) if k + 1 < len(parts)
                     else parts[k] for k in range(0, len(parts), 2)]
        return parts[0]

    def _smax(chunks):
        return _tree([jnp.max(c, axis=0, keepdims=True) for c in chunks],
                     jnp.maximum)

    s_blk = _tree([jnp.sum(jnp.exp(xs[j]), axis=0, keepdims=True)
                   for j in range(ns)], jnp.add)
    s_ref[0:1, :] = s_ref[0:1, :] + s_blk

    big = jnp.int32(2**30)
    bv, bi_ = [], []
    cbm = _smax(xs)
    for k in range(4):
        bi = _tree([jnp.min(jnp.where(xs[j] == cbm, vs[j], big), axis=0,
                            keepdims=True) for j in range(ns)],
                   jnp.minimum)
        bv.append(cbm)
        bi_.append(bi + i * _VB)
        if k < 3:
            xs = [jnp.where(vs[j] == bi, _NEG, xs[j]) for j in range(ns)]
            cbm = _smax(xs)

    h = []
    for k in range(4):
        hv, hi, _, _ = _ce(tv_ref[k:k + 1, :], ti_ref[k:k + 1, :],
                           bv[3 - k], bi_[3 - k])
        h.append((hv, hi))
    for a, b in ((0, 2), (1, 3), (0, 1), (2, 3)):
        hv, hi, lv, li = _ce(h[a][0], h[a][1], h[b][0], h[b][1])
        h[a] = (hv, hi)
        h[b] = (lv, li)
    for k in range(4):
        tv_ref[k:k + 1, :] = h[k][0]
        ti_ref[k:k + 1, :] = h[k][1]

    @pl.when(i == _NSTEPS - 1)
    def _():
        shift = jnp.log(s_ref[0:1, :])
        pad_v = jnp.full((4, _NROWS), _NEG, f32)
        pad_i = jnp.zeros((4, _NROWS), jnp.int32)
        vals_ref[...] = jnp.concatenate(
            [tv_ref[k:k + 1, :] - shift for k in range(4)] + [pad_v], axis=0)
        idx_ref[...] = jnp.concatenate(
            [ti_ref[k:k + 1, :] for k in range(4)] + [pad_i], axis=0)


def _merge_body(v_ref, ix_ref, p_ref, tok_ref, beam_ref):
    v = v_ref[...]
    vocab = ix_ref[...]
    lane = jax.lax.broadcasted_iota(jnp.int32, (_BATCH, 16), 1)
    combo = (lane // 4) * _V + vocab
    big = jnp.int32(2**31 - 1)
    ps, toks, beams = [], [], []
    for _ in range(4):
        bm = jnp.max(v, axis=1, keepdims=True)
        bc = jnp.min(jnp.where(v == bm, combo, big), axis=1, keepdims=True)
        ps.append(bm)
        toks.append(bc % _V)
        beams.append(bc // _V)
        v = jnp.where(combo == bc, _NEG, v)
    p_ref[...] = jnp.concatenate(ps, axis=1)
    tok_ref[...] = jnp.concatenate(toks, axis=1)
    beam_ref[...] = jnp.concatenate(beams, axis=1)


def kernel(logits, prev_tokens):
    xt = jnp.transpose(logits.reshape(_NROWS, _V))
    pt = prev_tokens.astype(jnp.int32)

    vals_t, idx_t = pl.pallas_call(
        _scan_body,
        grid=(_NSTEPS,),
        in_specs=[
            pl.BlockSpec((_VB, _NROWS), lambda i: (i, 0)),
            pl.BlockSpec(memory_space=pltpu.MemorySpace.SMEM),
        ],
        out_specs=[
            pl.BlockSpec((8, _NROWS), lambda i: (0, 0)),
            pl.BlockSpec((8, _NROWS), lambda i: (0, 0)),
        ],
        out_shape=[
            jax.ShapeDtypeStruct((8, _NROWS), jnp.float32),
            jax.ShapeDtypeStruct((8, _NROWS), jnp.int32),
        ],
        scratch_shapes=[
            pltpu.VMEM((8, _NROWS), jnp.float32),
            pltpu.VMEM((8, _NROWS), jnp.float32),
            pltpu.VMEM((8, _NROWS), jnp.int32),
        ],
        compiler_params=pltpu.CompilerParams(
            dimension_semantics=("arbitrary",)),
    )(xt, pt)

    cv = (vals_t[:4].reshape(4, _NBEAMS, _BATCH).transpose(2, 1, 0)
          .reshape(_BATCH, 16))
    ci = (idx_t[:4].reshape(4, _NBEAMS, _BATCH).transpose(2, 1, 0)
          .reshape(_BATCH, 16))

    probs, toks, beams = pl.pallas_call(
        _merge_body,
        out_shape=[
            jax.ShapeDtypeStruct((_BATCH, 4), jnp.float32),
            jax.ShapeDtypeStruct((_BATCH, 4), jnp.int32),
            jax.ShapeDtypeStruct((_BATCH, 4), jnp.int32),
        ],
    )(cv, ci)

    new_ids = toks.T.reshape(-1, 1)
    return new_ids, probs.T, beams.T


# speedup vs baseline: 5.4349x; 1.0012x over previous
import jax
import jax.numpy as jnp
from jax.experimental import pallas as pl
from jax.experimental.pallas import tpu as pltpu

_NBEAMS = 4
_BATCH = 32
_V = 100000
_EOS = 2
_NEG = -1e9
_NROWS = _NBEAMS * _BATCH
_VB = 25000
_NSTEPS = _V // _VB
_NS = 25
_SL = _VB // _NS


def _ce(va, ia, vb, ib):
    sw = (vb > va) | ((vb == va) & (ib < ia))
    return (jnp.where(sw, vb, va), jnp.where(sw, ib, ia),
            jnp.where(sw, va, vb), jnp.where(sw, ia, ib))


def _scan_body(xt_ref, pt_ref, vals_ref, idx_ref, s_ref, tv_ref, ti_ref):
    i = pl.program_id(0)
    f32 = jnp.float32

    @pl.when(i == 0)
    def _():
        xt_ref[_EOS:_EOS + 1, :] = jnp.full((1, _NROWS), _NEG, f32)
        s_ref[...] = jnp.zeros((8, _NROWS), f32)
        tv_ref[...] = jnp.full((8, _NROWS), _NEG, f32)
        ti_ref[...] = jnp.full((8, _NROWS), 2**30, jnp.int32)

    sub8 = jax.lax.broadcasted_iota(jnp.int32, (8, _NROWS), 0)
    lane = jax.lax.broadcasted_iota(jnp.int32, (8, _NROWS), 1)
    for r in range(_NBEAMS):
        lane_is_r = lane == r
        for j in range(_BATCH):
            pos = pt_ref[r, j] - i * _VB
            q = jnp.clip((pos // 8) * 8, 0, _VB - 8)
            q = pl.multiple_of(q, 8)
            rem = pos - q
            seg = xt_ref[pl.ds(q, 8), :]
            xt_ref[pl.ds(q, 8), :] = jnp.where(
                (sub8 == rem) & lane_is_r, _NEG, seg)

    vidx = jax.lax.broadcasted_iota(jnp.int32, (_VB, _NROWS), 0)

    ns, sl = _NS, _SL
    xs = [xt_ref[j * sl:(j + 1) * sl, :] for j in range(ns)]
    vs = [vidx[j * sl:(j + 1) * sl] for j in range(ns)]

    def _tree(parts, op):
        while len(parts) > 1:
            parts = [op(parts[k], parts[k + 1]) if k + 1 < len(parts)
                     else parts[k] for k in range(0, len(parts), 2)]
        return parts[0]

    def _smax(chunks):
        return _tree([jnp.max(c, axis=0, keepdims=True) for c in chunks],
                     jnp.maximum)

    s_blk = _tree([jnp.sum(jnp.exp(xs[j]), axis=0, keepdims=True)
                   for j in range(ns)], jnp.add)
    s_ref[0:1, :] = s_ref[0:1, :] + s_blk

    big = jnp.int32(2**30)
    bv, bi_ = [], []
    cbm = _smax(xs)
    for k in range(4):
        bi = _tree([jnp.min(jnp.where(xs[j] == cbm, vs[j], big), axis=0,
                            keepdims=True) for j in range(ns)],
                   jnp.minimum)
        bv.append(cbm)
        bi_.append(bi + i * _VB)
        if k < 3:
            xs = [jnp.where(vs[j] == bi, _NEG, xs[j]) for j in range(ns)]
            cbm = _smax(xs)

    h = []
    for k in range(4):
        hv, hi, _, _ = _ce(tv_ref[k:k + 1, :], ti_ref[k:k + 1, :],
                           bv[3 - k], bi_[3 - k])
        h.append((hv, hi))
    for a, b in ((0, 2), (1, 3), (0, 1), (2, 3)):
        hv, hi, lv, li = _ce(h[a][0], h[a][1], h[b][0], h[b][1])
        h[a] = (hv, hi)
        h[b] = (lv, li)
    for k in range(4):
        tv_ref[k:k + 1, :] = h[k][0]
        ti_ref[k:k + 1, :] = h[k][1]

    @pl.when(i == _NSTEPS - 1)
    def _():
        shift = jnp.log(s_ref[0:1, :])
        pad_v = jnp.full((4, _NROWS), _NEG, f32)
        pad_i = jnp.zeros((4, _NROWS), jnp.int32)
        vals_ref[...] = jnp.concatenate(
            [tv_ref[k:k + 1, :] - shift for k in range(4)] + [pad_v], axis=0)
        idx_ref[...] = jnp.concatenate(
            [ti_ref[k:k + 1, :] for k in range(4)] + [pad_i], axis=0)


def _merge_body(v_ref, ix_ref, p_ref, tok_ref, beam_ref):
    v = v_ref[...]
    vocab = ix_ref[...]
    lane = jax.lax.broadcasted_iota(jnp.int32, (_BATCH, 16), 1)
    combo = (lane // 4) * _V + vocab
    big = jnp.int32(2**31 - 1)
    ps, toks, beams = [], [], []
    for _ in range(4):
        bm = jnp.max(v, axis=1, keepdims=True)
        bc = jnp.min(jnp.where(v == bm, combo, big), axis=1, keepdims=True)
        ps.append(bm)
        toks.append(bc % _V)
        beams.append(bc // _V)
        v = jnp.where(combo == bc, _NEG, v)
    p_ref[...] = jnp.concatenate(ps, axis=1)
    tok_ref[...] = jnp.concatenate(toks, axis=1)
    beam_ref[...] = jnp.concatenate(beams, axis=1)


def kernel(logits, prev_tokens):
    xt = jnp.transpose(logits.reshape(_NROWS, _V))
    pt = prev_tokens.astype(jnp.int32)

    vals_t, idx_t = pl.pallas_call(
        _scan_body,
        grid=(_NSTEPS,),
        in_specs=[
            pl.BlockSpec((_VB, _NROWS), lambda i: (i, 0)),
            pl.BlockSpec(memory_space=pltpu.MemorySpace.SMEM),
        ],
        out_specs=[
            pl.BlockSpec((8, _NROWS), lambda i: (0, 0)),
            pl.BlockSpec((8, _NROWS), lambda i: (0, 0)),
        ],
        out_shape=[
            jax.ShapeDtypeStruct((8, _NROWS), jnp.float32),
            jax.ShapeDtypeStruct((8, _NROWS), jnp.int32),
        ],
        scratch_shapes=[
            pltpu.VMEM((8, _NROWS), jnp.float32),
            pltpu.VMEM((8, _NROWS), jnp.float32),
            pltpu.VMEM((8, _NROWS), jnp.int32),
        ],
        compiler_params=pltpu.CompilerParams(
            dimension_semantics=("arbitrary",)),
    )(xt, pt)

    cv = (vals_t[:4].reshape(4, _NBEAMS, _BATCH).transpose(2, 1, 0)
          .reshape(_BATCH, 16))
    ci = (idx_t[:4].reshape(4, _NBEAMS, _BATCH).transpose(2, 1, 0)
          .reshape(_BATCH, 16))

    probs, toks, beams = pl.pallas_call(
        _merge_body,
        out_shape=[
            jax.ShapeDtypeStruct((_BATCH, 4), jnp.float32),
            jax.ShapeDtypeStruct((_BATCH, 4), jnp.int32),
            jax.ShapeDtypeStruct((_BATCH, 4), jnp.int32),
        ],
    )(cv, ci)

    new_ids = toks.T.reshape(-1, 1)
    return new_ids, probs.T, beams.T


# speedup vs baseline: 5.7453x; 1.0571x over previous
import jax
import jax.numpy as jnp
from jax.experimental import pallas as pl
from jax.experimental.pallas import tpu as pltpu

_NBEAMS = 4
_BATCH = 32
_V = 100000
_EOS = 2
_NEG = -1e9
_NROWS = _NBEAMS * _BATCH
_VB = 25000
_NSTEPS = _V // _VB
_NS = 25
_SL = _VB // _NS


def _ce(va, ia, vb, ib):
    sw = (vb > va) | ((vb == va) & (ib < ia))
    return (jnp.where(sw, vb, va), jnp.where(sw, ib, ia),
            jnp.where(sw, va, vb), jnp.where(sw, ia, ib))


def _scan_body(xt_ref, pt_ref, p_ref, tok_ref, beam_ref, s_ref, tv_ref,
               ti_ref):
    i = pl.program_id(0)
    f32 = jnp.float32

    @pl.when(i == 0)
    def _():
        xt_ref[_EOS:_EOS + 1, :] = jnp.full((1, _NROWS), _NEG, f32)
        s_ref[...] = jnp.zeros((8, _NROWS), f32)
        tv_ref[...] = jnp.full((8, _NROWS), _NEG, f32)
        ti_ref[...] = jnp.full((8, _NROWS), 2**30, jnp.int32)

    sub8 = jax.lax.broadcasted_iota(jnp.int32, (8, _NROWS), 0)
    lane = jax.lax.broadcasted_iota(jnp.int32, (8, _NROWS), 1)
    for r in range(_NBEAMS):
        lane_is_r = lane == r
        for j in range(_BATCH):
            pos = pt_ref[r, j] - i * _VB
            q = jnp.clip((pos // 8) * 8, 0, _VB - 8)
            q = pl.multiple_of(q, 8)
            rem = pos - q
            seg = xt_ref[pl.ds(q, 8), :]
            xt_ref[pl.ds(q, 8), :] = jnp.where(
                (sub8 == rem) & lane_is_r, _NEG, seg)

    vidx = jax.lax.broadcasted_iota(jnp.int32, (_VB, _NROWS), 0)

    ns, sl = _NS, _SL
    xs = [xt_ref[j * sl:(j + 1) * sl, :] for j in range(ns)]
    vs = [vidx[j * sl:(j + 1) * sl] for j in range(ns)]

    def _tree(parts, op):
        while len(parts) > 1:
            parts = [op(parts[k], parts[k + 1]) if k + 1 < len(parts)
                     else parts[k] for k in range(0, len(parts), 2)]
        return parts[0]

    def _smax(chunks):
        return _tree([jnp.max(c, axis=0, keepdims=True) for c in chunks],
                     jnp.maximum)

    s_blk = _tree([jnp.sum(jnp.exp(xs[j]), axis=0, keepdims=True)
                   for j in range(ns)], jnp.add)
    s_ref[0:1, :] = s_ref[0:1, :] + s_blk

    big = jnp.int32(2**30)
    bv, bi_ = [], []
    cbm = _smax(xs)
    for k in range(4):
        bi = _tree([jnp.min(jnp.where(xs[j] == cbm, vs[j], big), axis=0,
                            keepdims=True) for j in range(ns)],
                   jnp.minimum)
        bv.append(cbm)
        bi_.append(bi + i * _VB)
        if k < 3:
            xs = [jnp.where(vs[j] == bi, _NEG, xs[j]) for j in range(ns)]
            cbm = _smax(xs)

    h = []
    for k in range(4):
        hv, hi, _, _ = _ce(tv_ref[k:k + 1, :], ti_ref[k:k + 1, :],
                           bv[3 - k], bi_[3 - k])
        h.append((hv, hi))
    for a, b in ((0, 2), (1, 3), (0, 1), (2, 3)):
        hv, hi, lv, li = _ce(h[a][0], h[a][1], h[b][0], h[b][1])
        h[a] = (hv, hi)
        h[b] = (lv, li)
    for k in range(4):
        tv_ref[k:k + 1, :] = h[k][0]
        ti_ref[k:k + 1, :] = h[k][1]

    @pl.when(i == _NSTEPS - 1)
    def _():
        big2 = jnp.int32(2**31 - 1)
        vals16, combo16 = [], []
        for b in range(_NBEAMS):
            sh = jnp.log(s_ref[0:1, b * _BATCH:(b + 1) * _BATCH])
            for c in range(4):
                vals16.append(
                    tv_ref[c:c + 1, b * _BATCH:(b + 1) * _BATCH] - sh)
                combo16.append(
                    ti_ref[c:c + 1, b * _BATCH:(b + 1) * _BATCH] + b * _V)
        ps, toks, beams = [], [], []
        for k in range(4):
            m = _tree(vals16, jnp.maximum)
            bc = _tree([jnp.where(vals16[t] == m, combo16[t], big2)
                        for t in range(16)], jnp.minimum)
            ps.append(m)
            toks.append(bc % _V)
            beams.append(bc // _V)
            if k < 3:
                vals16 = [jnp.where(combo16[t] == bc, _NEG, vals16[t])
                          for t in range(16)]
        pad = jnp.zeros((4, _BATCH), f32)
        padi = jnp.zeros((4, _BATCH), jnp.int32)
        p_ref[...] = jnp.concatenate(ps + [pad], axis=0)
        tok_ref[...] = jnp.concatenate(toks + [padi], axis=0)
        beam_ref[...] = jnp.concatenate(beams + [padi], axis=0)


def kernel(logits, prev_tokens):
    xt = jnp.transpose(logits.reshape(_NROWS, _V))
    pt = prev_tokens.astype(jnp.int32)

    p_out, t_out, b_out = pl.pallas_call(
        _scan_body,
        grid=(_NSTEPS,),
        in_specs=[
            pl.BlockSpec((_VB, _NROWS), lambda i: (i, 0)),
            pl.BlockSpec(memory_space=pltpu.MemorySpace.SMEM),
        ],
        out_specs=[
            pl.BlockSpec((8, _BATCH), lambda i: (0, 0)),
            pl.BlockSpec((8, _BATCH), lambda i: (0, 0)),
            pl.BlockSpec((8, _BATCH), lambda i: (0, 0)),
        ],
        out_shape=[
            jax.ShapeDtypeStruct((8, _BATCH), jnp.float32),
            jax.ShapeDtypeStruct((8, _BATCH), jnp.int32),
            jax.ShapeDtypeStruct((8, _BATCH), jnp.int32),
        ],
        scratch_shapes=[
            pltpu.VMEM((8, _NROWS), jnp.float32),
            pltpu.VMEM((8, _NROWS), jnp.float32),
            pltpu.VMEM((8, _NROWS), jnp.int32),
        ],
        compiler_params=pltpu.CompilerParams(
            dimension_semantics=("arbitrary",)),
    )(xt, pt)

    new_ids = t_out[:4].reshape(-1, 1)
    return new_ids, p_out[:4], b_out[:4]
